# baseline (speedup 1.0000x reference)
import functools

import jax
import jax.numpy as jnp
from jax import lax
from jax.experimental import pallas as pl
from jax.experimental.pallas import tpu as pltpu
from jax.experimental.pallas import tpu_sc as plsc

B = 16384
D = 64
LN = int(B * 0.4)
HEAD = B - LN
NC = 2
NS = 16
NW = NC * NS
BPW = B // NW
CH = 128
NCH = BPW // CH
L = 16
V = 100000
EI_OFF = V
NI_OFF = 2 * V

_MESH = plsc.VectorSubcoreMesh(
    core_axis_name="c", subcore_axis_name="s", num_cores=NC, num_subcores=NS)
_PARAMS = pltpu.CompilerParams(
    needs_layout_passes=False, use_tc_tiling_on_sc=False)

_f32 = jnp.float32
_i32 = jnp.int32


def _wid_base():
    wid = lax.axis_index("c") * NS + lax.axis_index("s")
    return wid, wid * BPW


def _k1_body(u_hbm, i_hbm, j_hbm, uf_hbm, cat_hbm,
             ni_out, s1_out, duj_out, g_out, l2p_out,
             uidx, iidx, jidx, j2pos, j2idx, iidxN, jidxE, j2idxE, j2idxN,
             gv, s1v, dujv,
             ubuf0, ibuf0, niebuf0, jbuf0, j2buf0, njebuf0, nibuf0,
             ubuf1, ibuf1, niebuf1, jbuf1, j2buf1, njebuf1, nibuf1,
             l2stage, sem0, sem1):
    wid, base = _wid_base()
    iota = lax.iota(_i32, L)
    bufs = [(ubuf0, ibuf0, niebuf0, jbuf0, j2buf0, njebuf0, nibuf0),
            (ubuf1, ibuf1, niebuf1, jbuf1, j2buf1, njebuf1, nibuf1)]
    sems = [sem0, sem1]

    for ch in range(NCH):
        off = base + ch * CH
        pltpu.sync_copy(u_hbm.at[pl.ds(off, CH)], uidx.at[ch])
        pltpu.sync_copy(i_hbm.at[pl.ds(off, CH)], iidx.at[ch])
        pltpu.sync_copy(j_hbm.at[pl.ds(off, CH)], jidx.at[ch])

    for ch in range(NCH):
        def fj(v, _, ch=ch):
            kv = jnp.full((L,), base + ch * CH, _i32) + v * L + iota
            j2pos[ch, pl.ds(v * L, L)] = jnp.where(kv < LN, kv + HEAD, kv - LN)
            return 0
        lax.fori_loop(0, CH // L, fj, 0)
        pltpu.async_copy(j_hbm.at[j2pos.at[ch]], j2idx.at[ch], sem0).wait()
        pltpu.async_copy(uf_hbm.at[uidx.at[ch]], gv.at[ch], sem0).wait()

    for ch in range(NCH):
        def sh(v, _, ch=ch):
            sl = pl.ds(v * L, L)
            iv = iidx[ch, sl]
            jv = jidx[ch, sl]
            j2v = j2idx[ch, sl]
            iidx[ch, sl] = iv + EI_OFF
            iidxN[ch, sl] = iv + NI_OFF
            jidxE[ch, sl] = jv + EI_OFF
            j2idxE[ch, sl] = j2v + EI_OFF
            j2idxN[ch, sl] = j2v + NI_OFF
            return 0
        lax.fori_loop(0, CH // L, sh, 0)

    def fire(ch, bi):
        ub, ib, neb, jb, j2b, njb, _ = bufs[bi]
        s = sems[bi]
        return [
            pltpu.async_copy(cat_hbm.at[uidx.at[ch]], ub, s),
            pltpu.async_copy(cat_hbm.at[iidx.at[ch]], ib, s),
            pltpu.async_copy(cat_hbm.at[iidxN.at[ch]], neb, s),
            pltpu.async_copy(cat_hbm.at[jidxE.at[ch]], jb, s),
            pltpu.async_copy(cat_hbm.at[j2idxE.at[ch]], j2b, s),
            pltpu.async_copy(cat_hbm.at[j2idxN.at[ch]], njb, s),
        ]

    l2acc = jnp.zeros((L,), _f32)
    cps = fire(0, 0)
    for ch in range(NCH):
        nxt = fire(ch + 1, (ch + 1) % 2) if ch + 1 < NCH else []
        for c in cps:
            c.wait()
        ub, ib, neb, jb, j2b, njb, nib = bufs[ch % 2]

        def row(r, carry, ub=ub, ib=ib, neb=neb, jb=jb, j2b=j2b, njb=njb,
                nib=nib, ch=ch):
            l2a, s1acc, dacc = carry
            kk = base + ch * CH + r
            sv = jnp.zeros((L,), _f32)
            dv = jnp.zeros((L,), _f32)
            for c in range(D // L):
                sl = pl.ds(c * L, L)
                uc = ub[r, sl]
                ic = ib[r, sl]
                jc = jb[r, sl]
                nic = jnp.clip(ic, -1.0, 1.0) + neb[r, sl]
                nib[r, sl] = nic
                addc = jnp.where(kk < HEAD, ic, nic)
                j2c = j2b[r, sl]
                nj2c = jnp.clip(j2c, -1.0, 1.0) + njb[r, sl]
                addjc = jnp.where(kk < LN, nj2c, j2c)
                sv = sv + uc * (addjc - addc)
                dv = dv + uc * jc
                l2a = l2a + uc * uc + addc * addc + jc * jc
            lane = jnp.bitwise_and(r, L - 1)
            s1acc = jnp.where(iota == lane, jnp.sum(sv), s1acc)
            dacc = jnp.where(iota == lane, jnp.sum(dv), dacc)

            @pl.when(lane == L - 1)
            def _():
                s1v[pl.ds(ch * CH + r - (L - 1), L)] = s1acc
                dujv[pl.ds(ch * CH + r - (L - 1), L)] = dacc

            return l2a, s1acc, dacc

        l2acc, _, _ = lax.fori_loop(
            0, CH, row,
            (l2acc, jnp.zeros((L,), _f32), jnp.zeros((L,), _f32)))
        pltpu.sync_copy(nib, ni_out.at[pl.ds(base + ch * CH, CH)])
        cps = nxt

    l2stage[...] = l2acc
    pltpu.sync_copy(l2stage, l2p_out.at[wid])
    pltpu.sync_copy(s1v, s1_out.at[pl.ds(base, BPW)])
    pltpu.sync_copy(dujv, duj_out.at[pl.ds(base, BPW)])
    for ch in range(NCH):
        pltpu.sync_copy(gv.at[ch], g_out.at[pl.ds(base + ch * CH, CH)])


_k1 = functools.partial(
    pl.kernel, _k1_body,
    out_type=[
        jax.ShapeDtypeStruct((B, D), _f32),
        jax.ShapeDtypeStruct((B,), _f32),
        jax.ShapeDtypeStruct((B,), _f32),
        jax.ShapeDtypeStruct((B,), _i32),
        jax.ShapeDtypeStruct((NW, L), _f32),
    ],
    mesh=_MESH,
    compiler_params=_PARAMS,
    scratch_types=[
        pltpu.VMEM((NCH, CH), _i32),
        pltpu.VMEM((NCH, CH), _i32),
        pltpu.VMEM((NCH, CH), _i32),
        pltpu.VMEM((NCH, CH), _i32),
        pltpu.VMEM((NCH, CH), _i32),
        pltpu.VMEM((NCH, CH), _i32),
        pltpu.VMEM((NCH, CH), _i32),
        pltpu.VMEM((NCH, CH), _i32),
        pltpu.VMEM((NCH, CH), _i32),
        pltpu.VMEM((NCH, CH), _i32),
        pltpu.VMEM((BPW,), _f32),
        pltpu.VMEM((BPW,), _f32),
    ] + [pltpu.VMEM((CH, D), _f32)] * 14 + [
        pltpu.VMEM((L,), _f32),
        pltpu.SemaphoreType.DMA,
        pltpu.SemaphoreType.DMA,
    ])()


def _k23_body(u_hbm, eu_hbm, g_hbm, duj_hbm, ni_hbm,
              s2_out, sqm_out, sqf_out,
              uidx, gall, pf, rankbuf, dujv, ppos, s2v,
              ubuf0, pnbuf0, ubuf1, pnbuf1,
              stm, stf, sem0, sem1):
    wid, base = _wid_base()
    iota = lax.iota(_i32, L)
    pltpu.sync_copy(g_hbm, gall)
    for ch in range(NCH):
        pltpu.sync_copy(u_hbm.at[pl.ds(base + ch * CH, CH)], uidx.at[ch])
    pltpu.sync_copy(duj_hbm.at[pl.ds(base, BPW)], dujv)

    myv0 = base // L

    def sweep(v, carry):
        mc, fc = carry
        g = gall[pl.ds(v * L, L)]
        kv = v * L + iota
        male = g == 1
        m32 = jnp.where(male, 1, 0)
        plsc.store_compressed(pf.at[pl.ds(B + mc, L)], kv, mask=male)
        plsc.store_compressed(pf.at[pl.ds(fc, L)], kv, mask=(g == 0))
        mexcl = plsc.cumsum(m32) - m32
        fexcl = iota - mexcl
        rank_vec = jnp.where(male, mc + mexcl, fc + fexcl)

        @pl.when((v >= myv0) & (v < myv0 + BPW // L))
        def _():
            rankbuf[pl.ds((v - myv0) * L, L)] = rank_vec

        pm = plsc.all_reduce_population_count(male)
        return mc + pm[0], fc + (L - pm[0])

    mc, _ = lax.fori_loop(0, B // L, sweep, (0, 0))
    M = mc
    Fm1 = jnp.full((L,), B - M - 1, _i32)
    Mm1 = jnp.full((L,), M - 1, _i32)

    def pidx(v, _):
        rv = rankbuf[pl.ds(v * L, L)]
        male = gall[pl.ds(base + v * L, L)] == 1
        pr = jnp.where(male, lax.rem(rv, Fm1), lax.rem(rv, Mm1))
        ppi = jnp.where(male, pr, B + pr)
        ppos[v // (CH // L), pl.ds((v % (CH // L)) * L, L)] = (
            plsc.load_gather(pf, [ppi]))
        return 0

    lax.fori_loop(0, BPW // L, pidx, 0)

    bufs = [(ubuf0, pnbuf0), (ubuf1, pnbuf1)]
    sems = [sem0, sem1]

    def fire(ch, bi):
        ub, pb = bufs[bi]
        s = sems[bi]
        return [pltpu.async_copy(eu_hbm.at[uidx.at[ch]], ub, s),
                pltpu.async_copy(ni_hbm.at[ppos.at[ch]], pb, s)]

    smacc = jnp.zeros((L,), _f32)
    sfacc = jnp.zeros((L,), _f32)
    cps = fire(0, 0)
    for ch in range(NCH):
        nxt = fire(ch + 1, (ch + 1) % 2) if ch + 1 < NCH else []
        for c in cps:
            c.wait()
        ub, pb = bufs[ch % 2]

        def grp(gi, carry, ub=ub, pb=pb, ch=ch):
            sm, sf = carry
            rbase = gi * L
            pacc = jnp.zeros((L,), _f32)
            sqacc = jnp.zeros((L,), _f32)
            for rr in range(L):
                r = rbase + rr
                pv = jnp.zeros((L,), _f32)
                sq = jnp.zeros((L,), _f32)
                for c in range(D // L):
                    sl = pl.ds(c * L, L)
                    uc = ub[r, sl]
                    pc = pb[r, sl]
                    pv = pv + uc * pc
                    sq = sq + pc * pc
                pacc = jnp.where(iota == rr, jnp.sum(pv), pacc)
                sqacc = jnp.where(iota == rr, jnp.sum(sq), sqacc)
            gb = pl.ds(ch * CH + rbase, L)
            s2v[gb] = dujv[gb] - pacc
            male = gall[pl.ds(base + ch * CH + rbase, L)] == 1
            sm = sm + jnp.where(male, sqacc, 0.0)
            sf = sf + jnp.where(male, 0.0, sqacc)
            return sm, sf

        smacc, sfacc = lax.fori_loop(0, CH // L, grp, (smacc, sfacc))
        cps = nxt

    stm[...] = smacc
    stf[...] = sfacc
    pltpu.sync_copy(stm, sqm_out.at[wid])
    pltpu.sync_copy(stf, sqf_out.at[wid])
    pltpu.sync_copy(s2v, s2_out.at[pl.ds(base, BPW)])


_k23 = functools.partial(
    pl.kernel, _k23_body,
    out_type=[
        jax.ShapeDtypeStruct((B,), _f32),
        jax.ShapeDtypeStruct((NW, L), _f32),
        jax.ShapeDtypeStruct((NW, L), _f32),
    ],
    mesh=_MESH,
    compiler_params=_PARAMS,
    scratch_types=[
        pltpu.VMEM((NCH, CH), _i32),
        pltpu.VMEM((B,), _i32),
        pltpu.VMEM((2 * B + 2 * L,), _i32),
        pltpu.VMEM((BPW,), _i32),
        pltpu.VMEM((BPW,), _f32),
        pltpu.VMEM((NCH, CH), _i32),
        pltpu.VMEM((BPW,), _f32),
        pltpu.VMEM((CH, D), _f32),
        pltpu.VMEM((CH, D), _f32),
        pltpu.VMEM((CH, D), _f32),
        pltpu.VMEM((CH, D), _f32),
        pltpu.VMEM((L,), _f32),
        pltpu.VMEM((L,), _f32),
        pltpu.SemaphoreType.DMA,
        pltpu.SemaphoreType.DMA,
    ])()


def _softplus(x):
    return jnp.maximum(x, 0.0) + jnp.log(1.0 + jnp.exp(-jnp.abs(x)))


def _k4_body(s1, s2, g, l2p, sqm, sqf, o1, o2, o3):
    loss_add = jnp.sum(_softplus(s1[...])) / B
    l2 = 0.01 * jnp.sum(l2p[...]) / B
    male = g[...] == 1
    M = jnp.sum(jnp.where(male, 1, 0))
    Mf = M.astype(_f32)
    Ff = (B - M).astype(_f32)
    sp2 = _softplus(s2[...])
    lf = (jnp.sum(jnp.where(male, sp2, 0.0)) / Mf
          + jnp.sum(jnp.where(male, 0.0, sp2)) / Ff)
    l22 = 0.01 * jnp.sum(sqm[...]) / Mf + 0.01 * jnp.sum(sqf[...]) / Ff
    o1[0, 0] = loss_add + l2
    o2[0, 0] = l2
    o3[0, 0] = lf + l22


_k4 = pl.pallas_call(
    _k4_body,
    out_shape=[jax.ShapeDtypeStruct((1, 1), _f32)] * 3,
    out_specs=[pl.BlockSpec(memory_space=pltpu.SMEM)] * 3,
)


@jax.jit
def kernel(adj_pos, u_batch, i_batch, j_batch, users_features,
           embed_user, embed_item, noise_item):
    del adj_pos
    cat = jnp.concatenate([embed_user, embed_item, noise_item], axis=0)
    ni, s1, duj, g, l2p = _k1(
        u_batch, i_batch, j_batch, users_features, cat)
    s2, sqm, sqf = _k23(u_batch, cat, g, duj, ni)
    o1, o2, o3 = _k4(
        s1.reshape(B // 128, 128), s2.reshape(B // 128, 128),
        g.reshape(B // 128, 128), l2p.reshape(NW * L // 128, 128),
        sqm.reshape(NW * L // 128, 128), sqf.reshape(NW * L // 128, 128))
    return o1[0, 0], o2[0, 0], o3[0, 0]


# speedup vs baseline: 1.5000x; 1.5000x over previous
import functools

import jax
import jax.numpy as jnp
from jax import lax
from jax.experimental import pallas as pl
from jax.experimental.pallas import tpu as pltpu
from jax.experimental.pallas import tpu_sc as plsc

B = 16384
D = 64
LN = int(B * 0.4)
HEAD = B - LN
NC = 2
NS = 16
NW = NC * NS
BPW = B // NW
CH = 128
NCH = BPW // CH
L = 16
V = 100000
EI_OFF = V
NI_OFF = 2 * V

_MESH = plsc.VectorSubcoreMesh(
    core_axis_name="c", subcore_axis_name="s", num_cores=NC, num_subcores=NS)
_PARAMS = pltpu.CompilerParams(
    needs_layout_passes=False, use_tc_tiling_on_sc=False)

_f32 = jnp.float32
_i32 = jnp.int32


def _wid_base():
    wid = lax.axis_index("c") * NS + lax.axis_index("s")
    return wid, wid * BPW


def _k1_body(u_hbm, i_hbm, j_hbm, uf_hbm, eu_hbm, ei_hbm, nit_hbm,
             ni_out, s1_out, duj_out, g_out, l2p_out,
             uidx, iidx, jidx, j2pos, j2idx, gv, s1v, dujv,
             ubuf0, ibuf0, niebuf0, jbuf0, j2buf0, njebuf0, nibuf0,
             ubuf1, ibuf1, niebuf1, jbuf1, j2buf1, njebuf1, nibuf1,
             l2stage, sem0, sem1):
    wid, base = _wid_base()
    iota = lax.iota(_i32, L)
    bufs = [(ubuf0, ibuf0, niebuf0, jbuf0, j2buf0, njebuf0, nibuf0),
            (ubuf1, ibuf1, niebuf1, jbuf1, j2buf1, njebuf1, nibuf1)]
    sems = [sem0, sem1]

    for ch in range(NCH):
        off = base + ch * CH
        pltpu.sync_copy(u_hbm.at[pl.ds(off, CH)], uidx.at[ch])
        pltpu.sync_copy(i_hbm.at[pl.ds(off, CH)], iidx.at[ch])
        pltpu.sync_copy(j_hbm.at[pl.ds(off, CH)], jidx.at[ch])

    for ch in range(NCH):
        def fj(v, _, ch=ch):
            kv = jnp.full((L,), base + ch * CH, _i32) + v * L + iota
            j2pos[ch, pl.ds(v * L, L)] = jnp.where(kv < LN, kv + HEAD, kv - LN)
            return 0
        lax.fori_loop(0, CH // L, fj, 0)
        pltpu.async_copy(j_hbm.at[j2pos.at[ch]], j2idx.at[ch], sem0).wait()
        pltpu.async_copy(uf_hbm.at[uidx.at[ch]], gv.at[ch], sem0).wait()

    def fire(ch, bi):
        ub, ib, neb, jb, j2b, njb, _ = bufs[bi]
        s = sems[bi]
        return [
            pltpu.async_copy(eu_hbm.at[uidx.at[ch]], ub, s),
            pltpu.async_copy(ei_hbm.at[iidx.at[ch]], ib, s),
            pltpu.async_copy(nit_hbm.at[iidx.at[ch]], neb, s),
            pltpu.async_copy(ei_hbm.at[jidx.at[ch]], jb, s),
            pltpu.async_copy(ei_hbm.at[j2idx.at[ch]], j2b, s),
            pltpu.async_copy(nit_hbm.at[j2idx.at[ch]], njb, s),
        ]

    l2acc = jnp.zeros((L,), _f32)
    cps = fire(0, 0)
    for ch in range(NCH):
        nxt = fire(ch + 1, (ch + 1) % 2) if ch + 1 < NCH else []
        for c in cps:
            c.wait()
        ub, ib, neb, jb, j2b, njb, nib = bufs[ch % 2]

        def row(r, carry, ub=ub, ib=ib, neb=neb, jb=jb, j2b=j2b, njb=njb,
                nib=nib, ch=ch):
            l2a, s1acc, dacc = carry
            kk = base + ch * CH + r
            sv = jnp.zeros((L,), _f32)
            dv = jnp.zeros((L,), _f32)
            for c in range(D // L):
                sl = pl.ds(c * L, L)
                uc = ub[r, sl]
                ic = ib[r, sl]
                jc = jb[r, sl]
                nic = jnp.clip(ic, -1.0, 1.0) + neb[r, sl]
                nib[r, sl] = nic
                addc = jnp.where(kk < HEAD, ic, nic)
                j2c = j2b[r, sl]
                nj2c = jnp.clip(j2c, -1.0, 1.0) + njb[r, sl]
                addjc = jnp.where(kk < LN, nj2c, j2c)
                sv = sv + uc * (addjc - addc)
                dv = dv + uc * jc
                l2a = l2a + uc * uc + addc * addc + jc * jc
            lane = jnp.bitwise_and(r, L - 1)
            s1acc = jnp.where(iota == lane, jnp.sum(sv), s1acc)
            dacc = jnp.where(iota == lane, jnp.sum(dv), dacc)

            @pl.when(lane == L - 1)
            def _():
                s1v[pl.ds(ch * CH + r - (L - 1), L)] = s1acc
                dujv[pl.ds(ch * CH + r - (L - 1), L)] = dacc

            return l2a, s1acc, dacc

        l2acc, _, _ = lax.fori_loop(
            0, CH, row,
            (l2acc, jnp.zeros((L,), _f32), jnp.zeros((L,), _f32)))
        pltpu.sync_copy(nib, ni_out.at[pl.ds(base + ch * CH, CH)])
        cps = nxt

    l2stage[...] = l2acc
    pltpu.sync_copy(l2stage, l2p_out.at[wid])
    pltpu.sync_copy(s1v, s1_out.at[pl.ds(base, BPW)])
    pltpu.sync_copy(dujv, duj_out.at[pl.ds(base, BPW)])
    for ch in range(NCH):
        pltpu.sync_copy(gv.at[ch], g_out.at[pl.ds(base + ch * CH, CH)])


_k1 = functools.partial(
    pl.kernel, _k1_body,
    out_type=[
        jax.ShapeDtypeStruct((B, D), _f32),
        jax.ShapeDtypeStruct((B,), _f32),
        jax.ShapeDtypeStruct((B,), _f32),
        jax.ShapeDtypeStruct((B,), _i32),
        jax.ShapeDtypeStruct((NW, L), _f32),
    ],
    mesh=_MESH,
    compiler_params=_PARAMS,
    scratch_types=[
        pltpu.VMEM((NCH, CH), _i32),
        pltpu.VMEM((NCH, CH), _i32),
        pltpu.VMEM((NCH, CH), _i32),
        pltpu.VMEM((NCH, CH), _i32),
        pltpu.VMEM((NCH, CH), _i32),
        pltpu.VMEM((NCH, CH), _i32),
        pltpu.VMEM((BPW,), _f32),
        pltpu.VMEM((BPW,), _f32),
    ] + [pltpu.VMEM((CH, D), _f32)] * 14 + [
        pltpu.VMEM((L,), _f32),
        pltpu.SemaphoreType.DMA,
        pltpu.SemaphoreType.DMA,
    ])()


def _k23_body(u_hbm, eu_hbm, g_hbm, duj_hbm, ni_hbm,
              s2_out, sqm_out, sqf_out,
              uidx, gall, pf, rankbuf, dujv, ppos, s2v,
              ubuf0, pnbuf0, ubuf1, pnbuf1,
              stm, stf, sem0, sem1):
    wid, base = _wid_base()
    iota = lax.iota(_i32, L)
    pltpu.sync_copy(g_hbm, gall)
    for ch in range(NCH):
        pltpu.sync_copy(u_hbm.at[pl.ds(base + ch * CH, CH)], uidx.at[ch])
    pltpu.sync_copy(duj_hbm.at[pl.ds(base, BPW)], dujv)

    myv0 = base // L

    def sweep(v, carry):
        mc, fc = carry
        g = gall[pl.ds(v * L, L)]
        kv = v * L + iota
        male = g == 1
        m32 = jnp.where(male, 1, 0)
        plsc.store_compressed(pf.at[pl.ds(B + mc, L)], kv, mask=male)
        plsc.store_compressed(pf.at[pl.ds(fc, L)], kv, mask=(g == 0))
        mexcl = plsc.cumsum(m32) - m32
        fexcl = iota - mexcl
        rank_vec = jnp.where(male, mc + mexcl, fc + fexcl)

        @pl.when((v >= myv0) & (v < myv0 + BPW // L))
        def _():
            rankbuf[pl.ds((v - myv0) * L, L)] = rank_vec

        pm = plsc.all_reduce_population_count(male)
        return mc + pm[0], fc + (L - pm[0])

    mc, _ = lax.fori_loop(0, B // L, sweep, (0, 0))
    M = mc
    Fm1 = jnp.full((L,), B - M - 1, _i32)
    Mm1 = jnp.full((L,), M - 1, _i32)

    def pidx(v, _):
        rv = rankbuf[pl.ds(v * L, L)]
        male = gall[pl.ds(base + v * L, L)] == 1
        pr = jnp.where(male, lax.rem(rv, Fm1), lax.rem(rv, Mm1))
        ppi = jnp.where(male, pr, B + pr)
        ppos[v // (CH // L), pl.ds((v % (CH // L)) * L, L)] = (
            plsc.load_gather(pf, [ppi]))
        return 0

    lax.fori_loop(0, BPW // L, pidx, 0)

    bufs = [(ubuf0, pnbuf0), (ubuf1, pnbuf1)]
    sems = [sem0, sem1]

    def fire(ch, bi):
        ub, pb = bufs[bi]
        s = sems[bi]
        return [pltpu.async_copy(eu_hbm.at[uidx.at[ch]], ub, s),
                pltpu.async_copy(ni_hbm.at[ppos.at[ch]], pb, s)]

    smacc = jnp.zeros((L,), _f32)
    sfacc = jnp.zeros((L,), _f32)
    cps = fire(0, 0)
    for ch in range(NCH):
        nxt = fire(ch + 1, (ch + 1) % 2) if ch + 1 < NCH else []
        for c in cps:
            c.wait()
        ub, pb = bufs[ch % 2]

        def grp(gi, carry, ub=ub, pb=pb, ch=ch):
            sm, sf = carry
            rbase = gi * L
            pacc = jnp.zeros((L,), _f32)
            sqacc = jnp.zeros((L,), _f32)
            for rr in range(L):
                r = rbase + rr
                pv = jnp.zeros((L,), _f32)
                sq = jnp.zeros((L,), _f32)
                for c in range(D // L):
                    sl = pl.ds(c * L, L)
                    uc = ub[r, sl]
                    pc = pb[r, sl]
                    pv = pv + uc * pc
                    sq = sq + pc * pc
                pacc = jnp.where(iota == rr, jnp.sum(pv), pacc)
                sqacc = jnp.where(iota == rr, jnp.sum(sq), sqacc)
            gb = pl.ds(ch * CH + rbase, L)
            s2v[gb] = dujv[gb] - pacc
            male = gall[pl.ds(base + ch * CH + rbase, L)] == 1
            sm = sm + jnp.where(male, sqacc, 0.0)
            sf = sf + jnp.where(male, 0.0, sqacc)
            return sm, sf

        smacc, sfacc = lax.fori_loop(0, CH // L, grp, (smacc, sfacc))
        cps = nxt

    stm[...] = smacc
    stf[...] = sfacc
    pltpu.sync_copy(stm, sqm_out.at[wid])
    pltpu.sync_copy(stf, sqf_out.at[wid])
    pltpu.sync_copy(s2v, s2_out.at[pl.ds(base, BPW)])


_k23 = functools.partial(
    pl.kernel, _k23_body,
    out_type=[
        jax.ShapeDtypeStruct((B,), _f32),
        jax.ShapeDtypeStruct((NW, L), _f32),
        jax.ShapeDtypeStruct((NW, L), _f32),
    ],
    mesh=_MESH,
    compiler_params=_PARAMS,
    scratch_types=[
        pltpu.VMEM((NCH, CH), _i32),
        pltpu.VMEM((B,), _i32),
        pltpu.VMEM((2 * B + 2 * L,), _i32),
        pltpu.VMEM((BPW,), _i32),
        pltpu.VMEM((BPW,), _f32),
        pltpu.VMEM((NCH, CH), _i32),
        pltpu.VMEM((BPW,), _f32),
        pltpu.VMEM((CH, D), _f32),
        pltpu.VMEM((CH, D), _f32),
        pltpu.VMEM((CH, D), _f32),
        pltpu.VMEM((CH, D), _f32),
        pltpu.VMEM((L,), _f32),
        pltpu.VMEM((L,), _f32),
        pltpu.SemaphoreType.DMA,
        pltpu.SemaphoreType.DMA,
    ])()


def _softplus(x):
    return jnp.maximum(x, 0.0) + jnp.log(1.0 + jnp.exp(-jnp.abs(x)))


def _k4_body(s1, s2, g, l2p, sqm, sqf, o1, o2, o3):
    loss_add = jnp.sum(_softplus(s1[...])) / B
    l2 = 0.01 * jnp.sum(l2p[...]) / B
    male = g[...] == 1
    M = jnp.sum(jnp.where(male, 1, 0))
    Mf = M.astype(_f32)
    Ff = (B - M).astype(_f32)
    sp2 = _softplus(s2[...])
    lf = (jnp.sum(jnp.where(male, sp2, 0.0)) / Mf
          + jnp.sum(jnp.where(male, 0.0, sp2)) / Ff)
    l22 = 0.01 * jnp.sum(sqm[...]) / Mf + 0.01 * jnp.sum(sqf[...]) / Ff
    o1[0, 0] = loss_add + l2
    o2[0, 0] = l2
    o3[0, 0] = lf + l22


_k4 = pl.pallas_call(
    _k4_body,
    out_shape=[jax.ShapeDtypeStruct((1, 1), _f32)] * 3,
    out_specs=[pl.BlockSpec(memory_space=pltpu.SMEM)] * 3,
)


@jax.jit
def kernel(adj_pos, u_batch, i_batch, j_batch, users_features,
           embed_user, embed_item, noise_item):
    del adj_pos
    ni, s1, duj, g, l2p = _k1(
        u_batch, i_batch, j_batch, users_features,
        embed_user, embed_item, noise_item)
    s2, sqm, sqf = _k23(u_batch, embed_user, g, duj, ni)
    o1, o2, o3 = _k4(
        s1.reshape(B // 128, 128), s2.reshape(B // 128, 128),
        g.reshape(B // 128, 128), l2p.reshape(NW * L // 128, 128),
        sqm.reshape(NW * L // 128, 128), sqf.reshape(NW * L // 128, 128))
    return o1[0, 0], o2[0, 0], o3[0, 0]


# speedup vs baseline: 1.5150x; 1.0100x over previous
import functools

import jax
import jax.numpy as jnp
from jax import lax
from jax.experimental import pallas as pl
from jax.experimental.pallas import tpu as pltpu
from jax.experimental.pallas import tpu_sc as plsc

B = 16384
D = 64
LN = int(B * 0.4)
HEAD = B - LN
NC = 2
NS = 16
NW = NC * NS
BPW = B // NW
CH = 128
NCH = BPW // CH
L = 16
V = 100000
EI_OFF = V
NI_OFF = 2 * V

_MESH = plsc.VectorSubcoreMesh(
    core_axis_name="c", subcore_axis_name="s", num_cores=NC, num_subcores=NS)
_PARAMS = pltpu.CompilerParams(
    needs_layout_passes=False, use_tc_tiling_on_sc=False)

_f32 = jnp.float32
_i32 = jnp.int32


def _wid_base():
    wid = lax.axis_index("c") * NS + lax.axis_index("s")
    return wid, wid * BPW


def _k1_body(u_hbm, i_hbm, j_hbm, uf_hbm, eu_hbm, ei_hbm, nit_hbm,
             ni_out, s1_out, duj_out, g_out, l2p_out, cnt_out,
             uidx, iidx, jidx, j2pos, j2idx, gv, s1v, dujv,
             ubuf0, ibuf0, niebuf0, jbuf0, j2buf0, njebuf0, nibuf0,
             ubuf1, ibuf1, niebuf1, jbuf1, j2buf1, njebuf1, nibuf1,
             l2stage, cntstage, sem0, sem1):
    wid, base = _wid_base()
    iota = lax.iota(_i32, L)
    bufs = [(ubuf0, ibuf0, niebuf0, jbuf0, j2buf0, njebuf0, nibuf0),
            (ubuf1, ibuf1, niebuf1, jbuf1, j2buf1, njebuf1, nibuf1)]
    sems = [sem0, sem1]

    for ch in range(NCH):
        off = base + ch * CH
        pltpu.sync_copy(u_hbm.at[pl.ds(off, CH)], uidx.at[ch])
        pltpu.sync_copy(i_hbm.at[pl.ds(off, CH)], iidx.at[ch])
        pltpu.sync_copy(j_hbm.at[pl.ds(off, CH)], jidx.at[ch])

    for ch in range(NCH):
        def fj(v, _, ch=ch):
            kv = jnp.full((L,), base + ch * CH, _i32) + v * L + iota
            j2pos[ch, pl.ds(v * L, L)] = jnp.where(kv < LN, kv + HEAD, kv - LN)
            return 0
        lax.fori_loop(0, CH // L, fj, 0)
        pltpu.async_copy(j_hbm.at[j2pos.at[ch]], j2idx.at[ch], sem0).wait()
        pltpu.async_copy(uf_hbm.at[uidx.at[ch]], gv.at[ch], sem0).wait()

    def fire(ch, bi):
        ub, ib, neb, jb, j2b, njb, _ = bufs[bi]
        s = sems[bi]
        return [
            pltpu.async_copy(eu_hbm.at[uidx.at[ch]], ub, s),
            pltpu.async_copy(ei_hbm.at[iidx.at[ch]], ib, s),
            pltpu.async_copy(nit_hbm.at[iidx.at[ch]], neb, s),
            pltpu.async_copy(ei_hbm.at[jidx.at[ch]], jb, s),
            pltpu.async_copy(ei_hbm.at[j2idx.at[ch]], j2b, s),
            pltpu.async_copy(nit_hbm.at[j2idx.at[ch]], njb, s),
        ]

    l2acc = jnp.zeros((L,), _f32)
    cps = fire(0, 0)
    for ch in range(NCH):
        nxt = fire(ch + 1, (ch + 1) % 2) if ch + 1 < NCH else []
        for c in cps:
            c.wait()
        ub, ib, neb, jb, j2b, njb, nib = bufs[ch % 2]

        def row(r, carry, ub=ub, ib=ib, neb=neb, jb=jb, j2b=j2b, njb=njb,
                nib=nib, ch=ch):
            l2a, s1acc, dacc = carry
            kk = base + ch * CH + r
            sv = jnp.zeros((L,), _f32)
            dv = jnp.zeros((L,), _f32)
            for c in range(D // L):
                sl = pl.ds(c * L, L)
                uc = ub[r, sl]
                ic = ib[r, sl]
                jc = jb[r, sl]
                nic = jnp.clip(ic, -1.0, 1.0) + neb[r, sl]
                nib[r, sl] = nic
                addc = jnp.where(kk < HEAD, ic, nic)
                j2c = j2b[r, sl]
                nj2c = jnp.clip(j2c, -1.0, 1.0) + njb[r, sl]
                addjc = jnp.where(kk < LN, nj2c, j2c)
                sv = sv + uc * (addjc - addc)
                dv = dv + uc * jc
                l2a = l2a + uc * uc + addc * addc + jc * jc
            lane = jnp.bitwise_and(r, L - 1)
            s1acc = jnp.where(iota == lane, jnp.sum(sv), s1acc)
            dacc = jnp.where(iota == lane, jnp.sum(dv), dacc)

            @pl.when(lane == L - 1)
            def _():
                s1v[pl.ds(ch * CH + r - (L - 1), L)] = s1acc
                dujv[pl.ds(ch * CH + r - (L - 1), L)] = dacc

            return l2a, s1acc, dacc

        l2acc, _, _ = lax.fori_loop(
            0, CH, row,
            (l2acc, jnp.zeros((L,), _f32), jnp.zeros((L,), _f32)))
        pltpu.sync_copy(nib, ni_out.at[pl.ds(base + ch * CH, CH)])
        cps = nxt

    cacc = jnp.zeros((L,), _i32)
    for ch in range(NCH):
        cacc = lax.fori_loop(
            0, CH // L,
            lambda v, a, ch=ch: a + gv[ch, pl.ds(v * L, L)], cacc)
    cntstage[...] = cacc
    pltpu.sync_copy(cntstage, cnt_out.at[wid])
    l2stage[...] = l2acc
    pltpu.sync_copy(l2stage, l2p_out.at[wid])
    pltpu.sync_copy(s1v, s1_out.at[pl.ds(base, BPW)])
    pltpu.sync_copy(dujv, duj_out.at[pl.ds(base, BPW)])
    for ch in range(NCH):
        pltpu.sync_copy(gv.at[ch], g_out.at[pl.ds(base + ch * CH, CH)])


_k1 = functools.partial(
    pl.kernel, _k1_body,
    out_type=[
        jax.ShapeDtypeStruct((B, D), _f32),
        jax.ShapeDtypeStruct((B,), _f32),
        jax.ShapeDtypeStruct((B,), _f32),
        jax.ShapeDtypeStruct((B,), _i32),
        jax.ShapeDtypeStruct((NW, L), _f32),
        jax.ShapeDtypeStruct((NW, L), _i32),
    ],
    mesh=_MESH,
    compiler_params=_PARAMS,
    scratch_types=[
        pltpu.VMEM((NCH, CH), _i32),
        pltpu.VMEM((NCH, CH), _i32),
        pltpu.VMEM((NCH, CH), _i32),
        pltpu.VMEM((NCH, CH), _i32),
        pltpu.VMEM((NCH, CH), _i32),
        pltpu.VMEM((NCH, CH), _i32),
        pltpu.VMEM((BPW,), _f32),
        pltpu.VMEM((BPW,), _f32),
    ] + [pltpu.VMEM((CH, D), _f32)] * 14 + [
        pltpu.VMEM((L,), _f32),
        pltpu.VMEM((L,), _i32),
        pltpu.SemaphoreType.DMA,
        pltpu.SemaphoreType.DMA,
    ])()


def _k23_body(u_hbm, eu_hbm, g_hbm, duj_hbm, ni_hbm, cnt_hbm,
              s2_out, sqm_out, sqf_out,
              uidx, gall, pf, rankbuf, dujv, ppos, s2v, cntv,
              ubuf0, pnbuf0, ubuf1, pnbuf1,
              stm, stf, sem0, sem1):
    wid, base = _wid_base()
    iota = lax.iota(_i32, L)
    pltpu.sync_copy(g_hbm, gall)
    for ch in range(NCH):
        pltpu.sync_copy(u_hbm.at[pl.ds(base + ch * CH, CH)], uidx.at[ch])
    pltpu.sync_copy(duj_hbm.at[pl.ds(base, BPW)], dujv)
    pltpu.sync_copy(cnt_hbm, cntv)

    def sweep(v, carry):
        mc, fc = carry
        g = gall[pl.ds(v * L, L)]
        kv = v * L + iota
        male = g == 1
        plsc.store_compressed(pf.at[pl.ds(B + mc, L)], kv, mask=male)
        plsc.store_compressed(pf.at[pl.ds(fc, L)], kv, mask=(g == 0))
        pm = plsc.all_reduce_population_count(male)
        return mc + pm[0], fc + (L - pm[0])

    mc, _ = lax.fori_loop(0, B // L, sweep, (0, 0))
    M = mc

    pacc = lax.fori_loop(
        0, NW,
        lambda w, a: a + jnp.where(w < wid, cntv[w, pl.ds(0, L)], 0),
        jnp.zeros((L,), _i32))
    mpre = jnp.full((L,), jnp.sum(pacc), _i32)

    def rnk(v, mp):
        g = gall[pl.ds(base + v * L, L)]
        male = g == 1
        m32 = jnp.where(male, 1, 0)
        mexcl = mp + plsc.cumsum(m32) - m32
        kv = base + v * L + iota
        rankbuf[pl.ds(v * L, L)] = jnp.where(male, mexcl, kv - mexcl)
        return mp + plsc.all_reduce_population_count(male)

    lax.fori_loop(0, BPW // L, rnk, mpre)
    Fm1 = jnp.full((L,), B - M - 1, _i32)
    Mm1 = jnp.full((L,), M - 1, _i32)

    def pidx(v, _):
        rv = rankbuf[pl.ds(v * L, L)]
        male = gall[pl.ds(base + v * L, L)] == 1
        pr = jnp.where(male, lax.rem(rv, Fm1), lax.rem(rv, Mm1))
        ppi = jnp.where(male, pr, B + pr)
        ppos[v // (CH // L), pl.ds((v % (CH // L)) * L, L)] = (
            plsc.load_gather(pf, [ppi]))
        return 0

    lax.fori_loop(0, BPW // L, pidx, 0)

    bufs = [(ubuf0, pnbuf0), (ubuf1, pnbuf1)]
    sems = [sem0, sem1]

    def fire(ch, bi):
        ub, pb = bufs[bi]
        s = sems[bi]
        return [pltpu.async_copy(eu_hbm.at[uidx.at[ch]], ub, s),
                pltpu.async_copy(ni_hbm.at[ppos.at[ch]], pb, s)]

    smacc = jnp.zeros((L,), _f32)
    sfacc = jnp.zeros((L,), _f32)
    cps = fire(0, 0)
    for ch in range(NCH):
        nxt = fire(ch + 1, (ch + 1) % 2) if ch + 1 < NCH else []
        for c in cps:
            c.wait()
        ub, pb = bufs[ch % 2]

        def grp(gi, carry, ub=ub, pb=pb, ch=ch):
            sm, sf = carry
            rbase = gi * L
            pacc = jnp.zeros((L,), _f32)
            sqacc = jnp.zeros((L,), _f32)
            for rr in range(L):
                r = rbase + rr
                pv = jnp.zeros((L,), _f32)
                sq = jnp.zeros((L,), _f32)
                for c in range(D // L):
                    sl = pl.ds(c * L, L)
                    uc = ub[r, sl]
                    pc = pb[r, sl]
                    pv = pv + uc * pc
                    sq = sq + pc * pc
                pacc = jnp.where(iota == rr, jnp.sum(pv), pacc)
                sqacc = jnp.where(iota == rr, jnp.sum(sq), sqacc)
            gb = pl.ds(ch * CH + rbase, L)
            s2v[gb] = dujv[gb] - pacc
            male = gall[pl.ds(base + ch * CH + rbase, L)] == 1
            sm = sm + jnp.where(male, sqacc, 0.0)
            sf = sf + jnp.where(male, 0.0, sqacc)
            return sm, sf

        smacc, sfacc = lax.fori_loop(0, CH // L, grp, (smacc, sfacc))
        cps = nxt

    stm[...] = smacc
    stf[...] = sfacc
    pltpu.sync_copy(stm, sqm_out.at[wid])
    pltpu.sync_copy(stf, sqf_out.at[wid])
    pltpu.sync_copy(s2v, s2_out.at[pl.ds(base, BPW)])


_k23 = functools.partial(
    pl.kernel, _k23_body,
    out_type=[
        jax.ShapeDtypeStruct((B,), _f32),
        jax.ShapeDtypeStruct((NW, L), _f32),
        jax.ShapeDtypeStruct((NW, L), _f32),
    ],
    mesh=_MESH,
    compiler_params=_PARAMS,
    scratch_types=[
        pltpu.VMEM((NCH, CH), _i32),
        pltpu.VMEM((B,), _i32),
        pltpu.VMEM((2 * B + 2 * L,), _i32),
        pltpu.VMEM((BPW,), _i32),
        pltpu.VMEM((BPW,), _f32),
        pltpu.VMEM((NCH, CH), _i32),
        pltpu.VMEM((BPW,), _f32),
        pltpu.VMEM((NW, L), _i32),
        pltpu.VMEM((CH, D), _f32),
        pltpu.VMEM((CH, D), _f32),
        pltpu.VMEM((CH, D), _f32),
        pltpu.VMEM((CH, D), _f32),
        pltpu.VMEM((L,), _f32),
        pltpu.VMEM((L,), _f32),
        pltpu.SemaphoreType.DMA,
        pltpu.SemaphoreType.DMA,
    ])()


def _softplus(x):
    return jnp.maximum(x, 0.0) + jnp.log(1.0 + jnp.exp(-jnp.abs(x)))


def _k4_body(s1, s2, g, l2p, sqm, sqf, o1, o2, o3):
    loss_add = jnp.sum(_softplus(s1[...])) / B
    l2 = 0.01 * jnp.sum(l2p[...]) / B
    male = g[...] == 1
    M = jnp.sum(jnp.where(male, 1, 0))
    Mf = M.astype(_f32)
    Ff = (B - M).astype(_f32)
    sp2 = _softplus(s2[...])
    lf = (jnp.sum(jnp.where(male, sp2, 0.0)) / Mf
          + jnp.sum(jnp.where(male, 0.0, sp2)) / Ff)
    l22 = 0.01 * jnp.sum(sqm[...]) / Mf + 0.01 * jnp.sum(sqf[...]) / Ff
    o1[0, 0] = loss_add + l2
    o2[0, 0] = l2
    o3[0, 0] = lf + l22


_k4 = pl.pallas_call(
    _k4_body,
    out_shape=[jax.ShapeDtypeStruct((1, 1), _f32)] * 3,
    out_specs=[pl.BlockSpec(memory_space=pltpu.SMEM)] * 3,
)


@jax.jit
def kernel(adj_pos, u_batch, i_batch, j_batch, users_features,
           embed_user, embed_item, noise_item):
    del adj_pos
    ni, s1, duj, g, l2p, cnt = _k1(
        u_batch, i_batch, j_batch, users_features,
        embed_user, embed_item, noise_item)
    s2, sqm, sqf = _k23(u_batch, embed_user, g, duj, ni, cnt)
    o1, o2, o3 = _k4(
        s1.reshape(B // 128, 128), s2.reshape(B // 128, 128),
        g.reshape(B // 128, 128), l2p.reshape(NW * L // 128, 128),
        sqm.reshape(NW * L // 128, 128), sqf.reshape(NW * L // 128, 128))
    return o1[0, 0], o2[0, 0], o3[0, 0]


# speedup vs baseline: 1.5919x; 1.0508x over previous
import functools

import jax
import jax.numpy as jnp
from jax import lax
from jax.experimental import pallas as pl
from jax.experimental.pallas import tpu as pltpu
from jax.experimental.pallas import tpu_sc as plsc

B = 16384
D = 64
LN = int(B * 0.4)
HEAD = B - LN
NC = 2
NS = 16
NW = NC * NS
BPW = B // NW
CH = 128
NCH = BPW // CH
L = 16
V = 100000
EI_OFF = V
NI_OFF = 2 * V

_MESH = plsc.VectorSubcoreMesh(
    core_axis_name="c", subcore_axis_name="s", num_cores=NC, num_subcores=NS)
_PARAMS = pltpu.CompilerParams(
    needs_layout_passes=False, use_tc_tiling_on_sc=False)

_f32 = jnp.float32
_i32 = jnp.int32


def _wid_base():
    wid = lax.axis_index("c") * NS + lax.axis_index("s")
    return wid, wid * BPW


def _k1_body(u_hbm, i_hbm, j_hbm, uf_hbm, eu_hbm, ei_hbm, nit_hbm,
             ni_out, s1_out, duj_out, g_out, l2p_out, cnt_out,
             uidx, iidx, jidx, j2pos, j2idx, gv, s1v, dujv,
             ubuf0, ibuf0, niebuf0, jbuf0, j2buf0, njebuf0, nibuf0,
             ubuf1, ibuf1, niebuf1, jbuf1, j2buf1, njebuf1, nibuf1,
             l2stage, cntstage, sem0, sem1):
    wid, base = _wid_base()
    iota = lax.iota(_i32, L)
    bufs = [(ubuf0, ibuf0, niebuf0, jbuf0, j2buf0, njebuf0, nibuf0),
            (ubuf1, ibuf1, niebuf1, jbuf1, j2buf1, njebuf1, nibuf1)]
    sems = [sem0, sem1]

    prep = []
    for ch in range(NCH):
        off = base + ch * CH
        prep.append(pltpu.async_copy(u_hbm.at[pl.ds(off, CH)], uidx.at[ch], sem0))
        prep.append(pltpu.async_copy(i_hbm.at[pl.ds(off, CH)], iidx.at[ch], sem0))
        prep.append(pltpu.async_copy(j_hbm.at[pl.ds(off, CH)], jidx.at[ch], sem0))

    for ch in range(NCH):
        def fj(v, _, ch=ch):
            kv = jnp.full((L,), base + ch * CH, _i32) + v * L + iota
            j2pos[ch, pl.ds(v * L, L)] = jnp.where(kv < LN, kv + HEAD, kv - LN)
            return 0
        lax.fori_loop(0, CH // L, fj, 0)
    for c in prep:
        c.wait()
    prep = []
    for ch in range(NCH):
        prep.append(pltpu.async_copy(j_hbm.at[j2pos.at[ch]], j2idx.at[ch], sem0))
        prep.append(pltpu.async_copy(uf_hbm.at[uidx.at[ch]], gv.at[ch], sem0))
    for c in prep:
        c.wait()

    def fire(ch, bi):
        ub, ib, neb, jb, j2b, njb, _ = bufs[bi]
        s = sems[bi]
        return [
            pltpu.async_copy(eu_hbm.at[uidx.at[ch]], ub, s),
            pltpu.async_copy(ei_hbm.at[iidx.at[ch]], ib, s),
            pltpu.async_copy(nit_hbm.at[iidx.at[ch]], neb, s),
            pltpu.async_copy(ei_hbm.at[jidx.at[ch]], jb, s),
            pltpu.async_copy(ei_hbm.at[j2idx.at[ch]], j2b, s),
            pltpu.async_copy(nit_hbm.at[j2idx.at[ch]], njb, s),
        ]

    l2acc = jnp.zeros((L,), _f32)
    cps = fire(0, 0)
    for ch in range(NCH):
        nxt = fire(ch + 1, (ch + 1) % 2) if ch + 1 < NCH else []
        for c in cps:
            c.wait()
        ub, ib, neb, jb, j2b, njb, nib = bufs[ch % 2]

        def row(r, carry, ub=ub, ib=ib, neb=neb, jb=jb, j2b=j2b, njb=njb,
                nib=nib, ch=ch):
            l2a, s1acc, dacc = carry
            kk = base + ch * CH + r
            sv = jnp.zeros((L,), _f32)
            dv = jnp.zeros((L,), _f32)
            for c in range(D // L):
                sl = pl.ds(c * L, L)
                uc = ub[r, sl]
                ic = ib[r, sl]
                jc = jb[r, sl]
                nic = jnp.clip(ic, -1.0, 1.0) + neb[r, sl]
                nib[r, sl] = nic
                addc = jnp.where(kk < HEAD, ic, nic)
                j2c = j2b[r, sl]
                nj2c = jnp.clip(j2c, -1.0, 1.0) + njb[r, sl]
                addjc = jnp.where(kk < LN, nj2c, j2c)
                sv = sv + uc * (addjc - addc)
                dv = dv + uc * jc
                l2a = l2a + uc * uc + addc * addc + jc * jc
            lane = jnp.bitwise_and(r, L - 1)
            s1acc = jnp.where(iota == lane, jnp.sum(sv), s1acc)
            dacc = jnp.where(iota == lane, jnp.sum(dv), dacc)

            @pl.when(lane == L - 1)
            def _():
                s1v[pl.ds(ch * CH + r - (L - 1), L)] = s1acc
                dujv[pl.ds(ch * CH + r - (L - 1), L)] = dacc

            return l2a, s1acc, dacc

        l2acc, _, _ = lax.fori_loop(
            0, CH, row,
            (l2acc, jnp.zeros((L,), _f32), jnp.zeros((L,), _f32)))
        pltpu.sync_copy(nib, ni_out.at[pl.ds(base + ch * CH, CH)])
        cps = nxt

    cacc = jnp.zeros((L,), _i32)
    for ch in range(NCH):
        cacc = lax.fori_loop(
            0, CH // L,
            lambda v, a, ch=ch: a + gv[ch, pl.ds(v * L, L)], cacc)
    cntstage[...] = cacc
    l2stage[...] = l2acc
    fin = [
        pltpu.async_copy(cntstage, cnt_out.at[wid], sem0),
        pltpu.async_copy(l2stage, l2p_out.at[wid], sem0),
        pltpu.async_copy(s1v, s1_out.at[pl.ds(base, BPW)], sem0),
        pltpu.async_copy(dujv, duj_out.at[pl.ds(base, BPW)], sem0),
    ] + [
        pltpu.async_copy(gv.at[ch], g_out.at[pl.ds(base + ch * CH, CH)], sem0)
        for ch in range(NCH)
    ]
    for c in fin:
        c.wait()


_k1 = functools.partial(
    pl.kernel, _k1_body,
    out_type=[
        jax.ShapeDtypeStruct((B, D), _f32),
        jax.ShapeDtypeStruct((B,), _f32),
        jax.ShapeDtypeStruct((B,), _f32),
        jax.ShapeDtypeStruct((B,), _i32),
        jax.ShapeDtypeStruct((NW, L), _f32),
        jax.ShapeDtypeStruct((NW, L), _i32),
    ],
    mesh=_MESH,
    compiler_params=_PARAMS,
    scratch_types=[
        pltpu.VMEM((NCH, CH), _i32),
        pltpu.VMEM((NCH, CH), _i32),
        pltpu.VMEM((NCH, CH), _i32),
        pltpu.VMEM((NCH, CH), _i32),
        pltpu.VMEM((NCH, CH), _i32),
        pltpu.VMEM((NCH, CH), _i32),
        pltpu.VMEM((BPW,), _f32),
        pltpu.VMEM((BPW,), _f32),
    ] + [pltpu.VMEM((CH, D), _f32)] * 14 + [
        pltpu.VMEM((L,), _f32),
        pltpu.VMEM((L,), _i32),
        pltpu.SemaphoreType.DMA,
        pltpu.SemaphoreType.DMA,
    ])()


def _k23_body(u_hbm, eu_hbm, g_hbm, duj_hbm, ni_hbm, cnt_hbm,
              s2_out, sqm_out, sqf_out,
              uidx, gall, pf, rankbuf, dujv, ppos, s2v, cntv,
              ubuf0, pnbuf0, ubuf1, pnbuf1,
              stm, stf, sem0, sem1):
    wid, base = _wid_base()
    iota = lax.iota(_i32, L)
    cg = pltpu.async_copy(g_hbm, gall, sem1)
    prep = [pltpu.async_copy(u_hbm.at[pl.ds(base + ch * CH, CH)],
                             uidx.at[ch], sem0) for ch in range(NCH)]
    prep.append(pltpu.async_copy(duj_hbm.at[pl.ds(base, BPW)], dujv, sem0))
    prep.append(pltpu.async_copy(cnt_hbm, cntv, sem0))
    cg.wait()

    def sweep(v, carry):
        mc, fc = carry
        g = gall[pl.ds(v * L, L)]
        kv = v * L + iota
        male = g == 1
        plsc.store_compressed(pf.at[pl.ds(B + mc, L)], kv, mask=male)
        plsc.store_compressed(pf.at[pl.ds(fc, L)], kv, mask=(g == 0))
        pm = plsc.all_reduce_population_count(male)
        return mc + pm[0], fc + (L - pm[0])

    mc, _ = lax.fori_loop(0, B // L, sweep, (0, 0))
    M = mc
    for c in prep:
        c.wait()

    pacc = lax.fori_loop(
        0, NW,
        lambda w, a: a + jnp.where(w < wid, cntv[w, pl.ds(0, L)], 0),
        jnp.zeros((L,), _i32))
    mpre = jnp.full((L,), jnp.sum(pacc), _i32)

    def rnk(v, mp):
        g = gall[pl.ds(base + v * L, L)]
        male = g == 1
        m32 = jnp.where(male, 1, 0)
        mexcl = mp + plsc.cumsum(m32) - m32
        kv = base + v * L + iota
        rankbuf[pl.ds(v * L, L)] = jnp.where(male, mexcl, kv - mexcl)
        return mp + plsc.all_reduce_population_count(male)

    lax.fori_loop(0, BPW // L, rnk, mpre)
    Fm1 = jnp.full((L,), B - M - 1, _i32)
    Mm1 = jnp.full((L,), M - 1, _i32)

    def pidx(v, _):
        rv = rankbuf[pl.ds(v * L, L)]
        male = gall[pl.ds(base + v * L, L)] == 1
        pr = jnp.where(male, lax.rem(rv, Fm1), lax.rem(rv, Mm1))
        ppi = jnp.where(male, pr, B + pr)
        ppos[v // (CH // L), pl.ds((v % (CH // L)) * L, L)] = (
            plsc.load_gather(pf, [ppi]))
        return 0

    lax.fori_loop(0, BPW // L, pidx, 0)

    bufs = [(ubuf0, pnbuf0), (ubuf1, pnbuf1)]
    sems = [sem0, sem1]

    def fire(ch, bi):
        ub, pb = bufs[bi]
        s = sems[bi]
        return [pltpu.async_copy(eu_hbm.at[uidx.at[ch]], ub, s),
                pltpu.async_copy(ni_hbm.at[ppos.at[ch]], pb, s)]

    smacc = jnp.zeros((L,), _f32)
    sfacc = jnp.zeros((L,), _f32)
    cps = fire(0, 0)
    for ch in range(NCH):
        nxt = fire(ch + 1, (ch + 1) % 2) if ch + 1 < NCH else []
        for c in cps:
            c.wait()
        ub, pb = bufs[ch % 2]

        def grp(gi, carry, ub=ub, pb=pb, ch=ch):
            sm, sf = carry
            rbase = gi * L
            pacc = jnp.zeros((L,), _f32)
            sqacc = jnp.zeros((L,), _f32)
            for rr in range(L):
                r = rbase + rr
                pv = jnp.zeros((L,), _f32)
                sq = jnp.zeros((L,), _f32)
                for c in range(D // L):
                    sl = pl.ds(c * L, L)
                    uc = ub[r, sl]
                    pc = pb[r, sl]
                    pv = pv + uc * pc
                    sq = sq + pc * pc
                pacc = jnp.where(iota == rr, jnp.sum(pv), pacc)
                sqacc = jnp.where(iota == rr, jnp.sum(sq), sqacc)
            gb = pl.ds(ch * CH + rbase, L)
            s2v[gb] = dujv[gb] - pacc
            male = gall[pl.ds(base + ch * CH + rbase, L)] == 1
            sm = sm + jnp.where(male, sqacc, 0.0)
            sf = sf + jnp.where(male, 0.0, sqacc)
            return sm, sf

        smacc, sfacc = lax.fori_loop(0, CH // L, grp, (smacc, sfacc))
        cps = nxt

    stm[...] = smacc
    stf[...] = sfacc
    fin = [
        pltpu.async_copy(stm, sqm_out.at[wid], sem0),
        pltpu.async_copy(stf, sqf_out.at[wid], sem0),
        pltpu.async_copy(s2v, s2_out.at[pl.ds(base, BPW)], sem0),
    ]
    for c in fin:
        c.wait()


_k23 = functools.partial(
    pl.kernel, _k23_body,
    out_type=[
        jax.ShapeDtypeStruct((B,), _f32),
        jax.ShapeDtypeStruct((NW, L), _f32),
        jax.ShapeDtypeStruct((NW, L), _f32),
    ],
    mesh=_MESH,
    compiler_params=_PARAMS,
    scratch_types=[
        pltpu.VMEM((NCH, CH), _i32),
        pltpu.VMEM((B,), _i32),
        pltpu.VMEM((2 * B + 2 * L,), _i32),
        pltpu.VMEM((BPW,), _i32),
        pltpu.VMEM((BPW,), _f32),
        pltpu.VMEM((NCH, CH), _i32),
        pltpu.VMEM((BPW,), _f32),
        pltpu.VMEM((NW, L), _i32),
        pltpu.VMEM((CH, D), _f32),
        pltpu.VMEM((CH, D), _f32),
        pltpu.VMEM((CH, D), _f32),
        pltpu.VMEM((CH, D), _f32),
        pltpu.VMEM((L,), _f32),
        pltpu.VMEM((L,), _f32),
        pltpu.SemaphoreType.DMA,
        pltpu.SemaphoreType.DMA,
    ])()


def _softplus(x):
    return jnp.maximum(x, 0.0) + jnp.log(1.0 + jnp.exp(-jnp.abs(x)))


def _k4_body(s1, s2, g, l2p, sqm, sqf, o1, o2, o3):
    loss_add = jnp.sum(_softplus(s1[...])) / B
    l2 = 0.01 * jnp.sum(l2p[...]) / B
    male = g[...] == 1
    M = jnp.sum(jnp.where(male, 1, 0))
    Mf = M.astype(_f32)
    Ff = (B - M).astype(_f32)
    sp2 = _softplus(s2[...])
    lf = (jnp.sum(jnp.where(male, sp2, 0.0)) / Mf
          + jnp.sum(jnp.where(male, 0.0, sp2)) / Ff)
    l22 = 0.01 * jnp.sum(sqm[...]) / Mf + 0.01 * jnp.sum(sqf[...]) / Ff
    o1[0, 0] = loss_add + l2
    o2[0, 0] = l2
    o3[0, 0] = lf + l22


_k4 = pl.pallas_call(
    _k4_body,
    out_shape=[jax.ShapeDtypeStruct((1, 1), _f32)] * 3,
    out_specs=[pl.BlockSpec(memory_space=pltpu.SMEM)] * 3,
)


@jax.jit
def kernel(adj_pos, u_batch, i_batch, j_batch, users_features,
           embed_user, embed_item, noise_item):
    del adj_pos
    ni, s1, duj, g, l2p, cnt = _k1(
        u_batch, i_batch, j_batch, users_features,
        embed_user, embed_item, noise_item)
    s2, sqm, sqf = _k23(u_batch, embed_user, g, duj, ni, cnt)
    o1, o2, o3 = _k4(
        s1.reshape(B // 128, 128), s2.reshape(B // 128, 128),
        g.reshape(B // 128, 128), l2p.reshape(NW * L // 128, 128),
        sqm.reshape(NW * L // 128, 128), sqf.reshape(NW * L // 128, 128))
    return o1[0, 0], o2[0, 0], o3[0, 0]


# speedup vs baseline: 1.6082x; 1.0103x over previous
import functools

import jax
import jax.numpy as jnp
from jax import lax
from jax.experimental import pallas as pl
from jax.experimental.pallas import tpu as pltpu
from jax.experimental.pallas import tpu_sc as plsc

B = 16384
D = 64
LN = int(B * 0.4)
HEAD = B - LN
NC = 2
NS = 16
NW = NC * NS
BPW = B // NW
CH = 128
NCH = BPW // CH
L = 16
V = 100000
EI_OFF = V
NI_OFF = 2 * V

_MESH = plsc.VectorSubcoreMesh(
    core_axis_name="c", subcore_axis_name="s", num_cores=NC, num_subcores=NS)
_PARAMS = pltpu.CompilerParams(
    needs_layout_passes=False, use_tc_tiling_on_sc=False)

_f32 = jnp.float32
_i32 = jnp.int32


def _wid_base():
    wid = lax.axis_index("c") * NS + lax.axis_index("s")
    return wid, wid * BPW


def _k1_body(u_hbm, i_hbm, j_hbm, uf_hbm, eu_hbm, ei_hbm, nit_hbm,
             ni_out, s1_out, duj_out, g_out, l2p_out, cnt_out,
             uidx, iidx, jidx, j2pos, j2idx, gv, s1v, dujv,
             ubuf0, ibuf0, niebuf0, jbuf0, j2buf0, njebuf0, nibuf0,
             ubuf1, ibuf1, niebuf1, jbuf1, j2buf1, njebuf1, nibuf1,
             l2stage, cntstage, sem0, sem1):
    wid, base = _wid_base()
    iota = lax.iota(_i32, L)
    bufs = [(ubuf0, ibuf0, niebuf0, jbuf0, j2buf0, njebuf0, nibuf0),
            (ubuf1, ibuf1, niebuf1, jbuf1, j2buf1, njebuf1, nibuf1)]
    sems = [sem0, sem1]

    prep = []
    for ch in range(NCH):
        off = base + ch * CH
        prep.append(pltpu.async_copy(u_hbm.at[pl.ds(off, CH)], uidx.at[ch], sem0))
        prep.append(pltpu.async_copy(i_hbm.at[pl.ds(off, CH)], iidx.at[ch], sem0))
        prep.append(pltpu.async_copy(j_hbm.at[pl.ds(off, CH)], jidx.at[ch], sem0))

    for ch in range(NCH):
        def fj(v, _, ch=ch):
            kv = jnp.full((L,), base + ch * CH, _i32) + v * L + iota
            j2pos[ch, pl.ds(v * L, L)] = jnp.where(kv < LN, kv + HEAD, kv - LN)
            return 0
        lax.fori_loop(0, CH // L, fj, 0)
    for c in prep:
        c.wait()
    prep = []
    for ch in range(NCH):
        prep.append(pltpu.async_copy(j_hbm.at[j2pos.at[ch]], j2idx.at[ch], sem0))
        prep.append(pltpu.async_copy(uf_hbm.at[uidx.at[ch]], gv.at[ch], sem0))
    for c in prep:
        c.wait()

    def fire(ch, bi):
        ub, ib, neb, jb, j2b, njb, _ = bufs[bi]
        s = sems[bi]
        return [
            pltpu.async_copy(eu_hbm.at[uidx.at[ch]], ub, s),
            pltpu.async_copy(ei_hbm.at[iidx.at[ch]], ib, s),
            pltpu.async_copy(nit_hbm.at[iidx.at[ch]], neb, s),
            pltpu.async_copy(ei_hbm.at[jidx.at[ch]], jb, s),
            pltpu.async_copy(ei_hbm.at[j2idx.at[ch]], j2b, s),
            pltpu.async_copy(nit_hbm.at[j2idx.at[ch]], njb, s),
        ]

    l2acc = jnp.zeros((L,), _f32)
    cps = fire(0, 0)
    for ch in range(NCH):
        nxt = fire(ch + 1, (ch + 1) % 2) if ch + 1 < NCH else []
        for c in cps:
            c.wait()
        ub, ib, neb, jb, j2b, njb, nib = bufs[ch % 2]

        def grp(gi, l2a, ub=ub, ib=ib, neb=neb, jb=jb, j2b=j2b, njb=njb,
                nib=nib, ch=ch):
            rbase = gi * L
            s1g = jnp.zeros((L,), _f32)
            dg = jnp.zeros((L,), _f32)
            for rr in range(L):
                r = rbase + rr
                kk = base + ch * CH + r
                sv = jnp.zeros((L,), _f32)
                dv = jnp.zeros((L,), _f32)
                for c in range(D // L):
                    sl = pl.ds(c * L, L)
                    uc = ub[r, sl]
                    ic = ib[r, sl]
                    jc = jb[r, sl]
                    nic = jnp.clip(ic, -1.0, 1.0) + neb[r, sl]
                    nib[r, sl] = nic
                    addc = jnp.where(kk < HEAD, ic, nic)
                    j2c = j2b[r, sl]
                    nj2c = jnp.clip(j2c, -1.0, 1.0) + njb[r, sl]
                    addjc = jnp.where(kk < LN, nj2c, j2c)
                    sv = sv + uc * (addjc - addc)
                    dv = dv + uc * jc
                    l2a = l2a + uc * uc + addc * addc + jc * jc
                s1g = jnp.where(iota == rr, jnp.sum(sv), s1g)
                dg = jnp.where(iota == rr, jnp.sum(dv), dg)
            gb = pl.ds(ch * CH + rbase, L)
            s1v[gb] = s1g
            dujv[gb] = dg
            return l2a

        l2acc = lax.fori_loop(0, CH // L, grp, l2acc)
        pltpu.sync_copy(nib, ni_out.at[pl.ds(base + ch * CH, CH)])
        cps = nxt

    cacc = jnp.zeros((L,), _i32)
    for ch in range(NCH):
        cacc = lax.fori_loop(
            0, CH // L,
            lambda v, a, ch=ch: a + gv[ch, pl.ds(v * L, L)], cacc)
    cntstage[...] = cacc
    l2stage[...] = l2acc
    fin = [
        pltpu.async_copy(cntstage, cnt_out.at[wid], sem0),
        pltpu.async_copy(l2stage, l2p_out.at[wid], sem0),
        pltpu.async_copy(s1v, s1_out.at[pl.ds(base, BPW)], sem0),
        pltpu.async_copy(dujv, duj_out.at[pl.ds(base, BPW)], sem0),
    ] + [
        pltpu.async_copy(gv.at[ch], g_out.at[pl.ds(base + ch * CH, CH)], sem0)
        for ch in range(NCH)
    ]
    for c in fin:
        c.wait()


_k1 = functools.partial(
    pl.kernel, _k1_body,
    out_type=[
        jax.ShapeDtypeStruct((B, D), _f32),
        jax.ShapeDtypeStruct((B,), _f32),
        jax.ShapeDtypeStruct((B,), _f32),
        jax.ShapeDtypeStruct((B,), _i32),
        jax.ShapeDtypeStruct((NW, L), _f32),
        jax.ShapeDtypeStruct((NW, L), _i32),
    ],
    mesh=_MESH,
    compiler_params=_PARAMS,
    scratch_types=[
        pltpu.VMEM((NCH, CH), _i32),
        pltpu.VMEM((NCH, CH), _i32),
        pltpu.VMEM((NCH, CH), _i32),
        pltpu.VMEM((NCH, CH), _i32),
        pltpu.VMEM((NCH, CH), _i32),
        pltpu.VMEM((NCH, CH), _i32),
        pltpu.VMEM((BPW,), _f32),
        pltpu.VMEM((BPW,), _f32),
    ] + [pltpu.VMEM((CH, D), _f32)] * 14 + [
        pltpu.VMEM((L,), _f32),
        pltpu.VMEM((L,), _i32),
        pltpu.SemaphoreType.DMA,
        pltpu.SemaphoreType.DMA,
    ])()


def _k23_body(u_hbm, eu_hbm, g_hbm, duj_hbm, ni_hbm, cnt_hbm,
              s2_out, sqm_out, sqf_out,
              uidx, gall, pf, rankbuf, dujv, ppos, s2v, cntv,
              ubuf0, pnbuf0, ubuf1, pnbuf1,
              stm, stf, sem0, sem1):
    wid, base = _wid_base()
    iota = lax.iota(_i32, L)
    cg = pltpu.async_copy(g_hbm, gall, sem1)
    prep = [pltpu.async_copy(u_hbm.at[pl.ds(base + ch * CH, CH)],
                             uidx.at[ch], sem0) for ch in range(NCH)]
    prep.append(pltpu.async_copy(duj_hbm.at[pl.ds(base, BPW)], dujv, sem0))
    prep.append(pltpu.async_copy(cnt_hbm, cntv, sem0))
    cg.wait()

    def sweep(v, carry):
        mc, fc = carry
        g = gall[pl.ds(v * L, L)]
        kv = v * L + iota
        male = g == 1
        plsc.store_compressed(pf.at[pl.ds(B + mc, L)], kv, mask=male)
        plsc.store_compressed(pf.at[pl.ds(fc, L)], kv, mask=(g == 0))
        pm = plsc.all_reduce_population_count(male)
        return mc + pm[0], fc + (L - pm[0])

    mc, _ = lax.fori_loop(0, B // L, sweep, (0, 0))
    M = mc
    for c in prep:
        c.wait()

    pacc = lax.fori_loop(
        0, NW,
        lambda w, a: a + jnp.where(w < wid, cntv[w, pl.ds(0, L)], 0),
        jnp.zeros((L,), _i32))
    mpre = jnp.full((L,), jnp.sum(pacc), _i32)

    def rnk(v, mp):
        g = gall[pl.ds(base + v * L, L)]
        male = g == 1
        m32 = jnp.where(male, 1, 0)
        mexcl = mp + plsc.cumsum(m32) - m32
        kv = base + v * L + iota
        rankbuf[pl.ds(v * L, L)] = jnp.where(male, mexcl, kv - mexcl)
        return mp + plsc.all_reduce_population_count(male)

    lax.fori_loop(0, BPW // L, rnk, mpre)
    Fm1 = jnp.full((L,), B - M - 1, _i32)
    Mm1 = jnp.full((L,), M - 1, _i32)

    def pidx(v, _):
        rv = rankbuf[pl.ds(v * L, L)]
        male = gall[pl.ds(base + v * L, L)] == 1
        pr = jnp.where(male, lax.rem(rv, Fm1), lax.rem(rv, Mm1))
        ppi = jnp.where(male, pr, B + pr)
        ppos[v // (CH // L), pl.ds((v % (CH // L)) * L, L)] = (
            plsc.load_gather(pf, [ppi]))
        return 0

    lax.fori_loop(0, BPW // L, pidx, 0)

    bufs = [(ubuf0, pnbuf0), (ubuf1, pnbuf1)]
    sems = [sem0, sem1]

    def fire(ch, bi):
        ub, pb = bufs[bi]
        s = sems[bi]
        return [pltpu.async_copy(eu_hbm.at[uidx.at[ch]], ub, s),
                pltpu.async_copy(ni_hbm.at[ppos.at[ch]], pb, s)]

    smacc = jnp.zeros((L,), _f32)
    sfacc = jnp.zeros((L,), _f32)
    cps = fire(0, 0)
    for ch in range(NCH):
        nxt = fire(ch + 1, (ch + 1) % 2) if ch + 1 < NCH else []
        for c in cps:
            c.wait()
        ub, pb = bufs[ch % 2]

        def grp(gi, carry, ub=ub, pb=pb, ch=ch):
            sm, sf = carry
            rbase = gi * L
            pacc = jnp.zeros((L,), _f32)
            sqacc = jnp.zeros((L,), _f32)
            for rr in range(L):
                r = rbase + rr
                pv = jnp.zeros((L,), _f32)
                sq = jnp.zeros((L,), _f32)
                for c in range(D // L):
                    sl = pl.ds(c * L, L)
                    uc = ub[r, sl]
                    pc = pb[r, sl]
                    pv = pv + uc * pc
                    sq = sq + pc * pc
                pacc = jnp.where(iota == rr, jnp.sum(pv), pacc)
                sqacc = jnp.where(iota == rr, jnp.sum(sq), sqacc)
            gb = pl.ds(ch * CH + rbase, L)
            s2v[gb] = dujv[gb] - pacc
            male = gall[pl.ds(base + ch * CH + rbase, L)] == 1
            sm = sm + jnp.where(male, sqacc, 0.0)
            sf = sf + jnp.where(male, 0.0, sqacc)
            return sm, sf

        smacc, sfacc = lax.fori_loop(0, CH // L, grp, (smacc, sfacc))
        cps = nxt

    stm[...] = smacc
    stf[...] = sfacc
    fin = [
        pltpu.async_copy(stm, sqm_out.at[wid], sem0),
        pltpu.async_copy(stf, sqf_out.at[wid], sem0),
        pltpu.async_copy(s2v, s2_out.at[pl.ds(base, BPW)], sem0),
    ]
    for c in fin:
        c.wait()


_k23 = functools.partial(
    pl.kernel, _k23_body,
    out_type=[
        jax.ShapeDtypeStruct((B,), _f32),
        jax.ShapeDtypeStruct((NW, L), _f32),
        jax.ShapeDtypeStruct((NW, L), _f32),
    ],
    mesh=_MESH,
    compiler_params=_PARAMS,
    scratch_types=[
        pltpu.VMEM((NCH, CH), _i32),
        pltpu.VMEM((B,), _i32),
        pltpu.VMEM((2 * B + 2 * L,), _i32),
        pltpu.VMEM((BPW,), _i32),
        pltpu.VMEM((BPW,), _f32),
        pltpu.VMEM((NCH, CH), _i32),
        pltpu.VMEM((BPW,), _f32),
        pltpu.VMEM((NW, L), _i32),
        pltpu.VMEM((CH, D), _f32),
        pltpu.VMEM((CH, D), _f32),
        pltpu.VMEM((CH, D), _f32),
        pltpu.VMEM((CH, D), _f32),
        pltpu.VMEM((L,), _f32),
        pltpu.VMEM((L,), _f32),
        pltpu.SemaphoreType.DMA,
        pltpu.SemaphoreType.DMA,
    ])()


def _softplus(x):
    return jnp.maximum(x, 0.0) + jnp.log(1.0 + jnp.exp(-jnp.abs(x)))


def _k4_body(s1, s2, g, l2p, sqm, sqf, o1, o2, o3):
    loss_add = jnp.sum(_softplus(s1[...])) / B
    l2 = 0.01 * jnp.sum(l2p[...]) / B
    male = g[...] == 1
    M = jnp.sum(jnp.where(male, 1, 0))
    Mf = M.astype(_f32)
    Ff = (B - M).astype(_f32)
    sp2 = _softplus(s2[...])
    lf = (jnp.sum(jnp.where(male, sp2, 0.0)) / Mf
          + jnp.sum(jnp.where(male, 0.0, sp2)) / Ff)
    l22 = 0.01 * jnp.sum(sqm[...]) / Mf + 0.01 * jnp.sum(sqf[...]) / Ff
    o1[0, 0] = loss_add + l2
    o2[0, 0] = l2
    o3[0, 0] = lf + l22


_k4 = pl.pallas_call(
    _k4_body,
    out_shape=[jax.ShapeDtypeStruct((1, 1), _f32)] * 3,
    out_specs=[pl.BlockSpec(memory_space=pltpu.SMEM)] * 3,
)


@jax.jit
def kernel(adj_pos, u_batch, i_batch, j_batch, users_features,
           embed_user, embed_item, noise_item):
    del adj_pos
    ni, s1, duj, g, l2p, cnt = _k1(
        u_batch, i_batch, j_batch, users_features,
        embed_user, embed_item, noise_item)
    s2, sqm, sqf = _k23(u_batch, embed_user, g, duj, ni, cnt)
    o1, o2, o3 = _k4(
        s1.reshape(B // 128, 128), s2.reshape(B // 128, 128),
        g.reshape(B // 128, 128), l2p.reshape(NW * L // 128, 128),
        sqm.reshape(NW * L // 128, 128), sqf.reshape(NW * L // 128, 128))
    return o1[0, 0], o2[0, 0], o3[0, 0]
